# Initial kernel scaffold; baseline (speedup 1.0000x reference)
import jax
import jax.numpy as jnp
from jax.experimental import pallas as pl


def kernel(x_nchw, weight_oihw, bias):
    raise NotImplementedError("write your pallas kernel here")


# speedup vs baseline: 3.5944x; 3.5944x over previous
import math

import jax
import jax.numpy as jnp
from jax import lax
from jax.experimental import pallas as pl
from jax.experimental.pallas import tpu as pltpu


def _gelu_exact(x):
    return 0.5 * x * (1.0 + lax.erf(x * (1.0 / math.sqrt(2.0))))


def _conv_body(x_ref, a_ref, b_ref, o_ref):
    C, H, W = x_ref.shape[1], x_ref.shape[2], x_ref.shape[3]
    x = x_ref[0].reshape(C * H, W).astype(jnp.bfloat16)
    zcol = jnp.zeros((C * H, 1), jnp.bfloat16)
    x_m = jnp.concatenate([zcol, x[:, : W - 1]], axis=1)
    x_p = jnp.concatenate([x[:, 1:], zcol], axis=1)
    acc = jnp.dot(a_ref[0], x_m, preferred_element_type=jnp.float32)
    acc = acc + jnp.dot(a_ref[1], x, preferred_element_type=jnp.float32)
    acc = acc + jnp.dot(a_ref[2], x_p, preferred_element_type=jnp.float32)
    y = acc.reshape(C, H, W) + b_ref[...][:, None, :]
    o_ref[0] = _gelu_exact(y).astype(o_ref.dtype)


def _build_tap_mats(weight_oihw, H):
    C = weight_oihw.shape[0]
    w = weight_oihw.astype(jnp.float32)
    mats = []
    for kx in range(3):
        m = jnp.zeros((C * H, C * H), jnp.float32)
        for ky in range(3):
            eye = jnp.eye(H, H, k=ky - 1, dtype=jnp.float32)
            m = m + jnp.einsum("oi,hp->ohip", w[:, :, ky, kx], eye).reshape(
                C * H, C * H
            )
        mats.append(m)
    return jnp.stack(mats).astype(jnp.bfloat16)


def _invariant(block_shape):
    index_map = lambda i: (0,) * len(block_shape)
    if hasattr(pl, "Buffered"):
        try:
            return pl.BlockSpec(block_shape, index_map, pipeline_mode=pl.Buffered(1))
        except TypeError:
            pass
    return pl.BlockSpec(block_shape, index_map)


def kernel(x_nchw, weight_oihw, bias):
    N, C, H, W = x_nchw.shape
    CH = C * H
    a_mats = _build_tap_mats(weight_oihw, H)
    bias_mat = jnp.broadcast_to(bias.astype(jnp.float32)[:, None], (C, W))

    return pl.pallas_call(
        _conv_body,
        out_shape=jax.ShapeDtypeStruct((N, C, H, W), x_nchw.dtype),
        grid=(N,),
        in_specs=[
            pl.BlockSpec((1, C, H, W), lambda i: (i, 0, 0, 0)),
            _invariant((3, CH, CH)),
            _invariant((C, W)),
        ],
        out_specs=pl.BlockSpec((1, C, H, W), lambda i: (i, 0, 0, 0)),
        compiler_params=pltpu.CompilerParams(
            dimension_semantics=("parallel",),
            vmem_limit_bytes=48 * 1024 * 1024,
        ),
    )(x_nchw, a_mats, bias_mat)


# speedup vs baseline: 6.6880x; 1.8607x over previous
import math

import jax
import jax.numpy as jnp
from jax import lax
from jax.experimental import pallas as pl
from jax.experimental.pallas import tpu as pltpu


def _gelu_exact(x):
    return 0.5 * x * (1.0 + lax.erf(x * (1.0 / math.sqrt(2.0))))


def _conv_body(x_ref, a_ref, b_ref, o_ref):
    C, H, W = x_ref.shape[1], x_ref.shape[2], x_ref.shape[3]
    CH = C * H
    xa = x_ref[0].reshape(CH, W).astype(jnp.bfloat16)
    xb = x_ref[1].reshape(CH, W).astype(jnp.bfloat16)
    zcol = jnp.zeros((CH, 1), jnp.bfloat16)
    x = jnp.concatenate([xa, xb], axis=1)
    x_m = jnp.concatenate(
        [zcol, xa[:, : W - 1], zcol, xb[:, : W - 1]], axis=1
    )
    x_p = jnp.concatenate(
        [xa[:, 1:], zcol, xb[:, 1:], zcol], axis=1
    )
    acc = jnp.dot(a_ref[0], x_m, preferred_element_type=jnp.float32)
    acc = acc + jnp.dot(a_ref[1], x, preferred_element_type=jnp.float32)
    acc = acc + jnp.dot(a_ref[2], x_p, preferred_element_type=jnp.float32)
    y = _gelu_exact(acc.reshape(C, H, 2 * W) + b_ref[...][:, None, :])
    o_ref[0] = y[:, :, :W].astype(o_ref.dtype)
    o_ref[1] = y[:, :, W:].astype(o_ref.dtype)


def _build_tap_mats(weight_oihw, H):
    C = weight_oihw.shape[0]
    w = weight_oihw.astype(jnp.float32)
    mats = []
    for kx in range(3):
        m = jnp.zeros((C * H, C * H), jnp.float32)
        for ky in range(3):
            eye = jnp.eye(H, H, k=ky - 1, dtype=jnp.float32)
            m = m + jnp.einsum("oi,hp->ohip", w[:, :, ky, kx], eye).reshape(
                C * H, C * H
            )
        mats.append(m)
    return jnp.stack(mats).astype(jnp.bfloat16)


def _invariant(block_shape):
    index_map = lambda i: (0,) * len(block_shape)
    if hasattr(pl, "Buffered"):
        try:
            return pl.BlockSpec(block_shape, index_map, pipeline_mode=pl.Buffered(1))
        except TypeError:
            pass
    return pl.BlockSpec(block_shape, index_map)


def kernel(x_nchw, weight_oihw, bias):
    N, C, H, W = x_nchw.shape
    CH = C * H
    a_mats = _build_tap_mats(weight_oihw, H)
    bias_mat = jnp.broadcast_to(bias.astype(jnp.float32)[:, None], (C, 2 * W))

    return pl.pallas_call(
        _conv_body,
        out_shape=jax.ShapeDtypeStruct((N, C, H, W), x_nchw.dtype),
        grid=(N // 2,),
        in_specs=[
            pl.BlockSpec((2, C, H, W), lambda i: (i, 0, 0, 0)),
            _invariant((3, CH, CH)),
            _invariant((C, 2 * W)),
        ],
        out_specs=pl.BlockSpec((2, C, H, W), lambda i: (i, 0, 0, 0)),
        compiler_params=pltpu.CompilerParams(
            dimension_semantics=("parallel",),
            vmem_limit_bytes=48 * 1024 * 1024,
        ),
    )(x_nchw, a_mats, bias_mat)


# speedup vs baseline: 7.9025x; 1.1816x over previous
import math

import jax
import jax.numpy as jnp
from jax import lax
from jax.experimental import pallas as pl
from jax.experimental.pallas import tpu as pltpu


def _gelu_exact(x):
    return 0.5 * x * (1.0 + lax.erf(x * (1.0 / math.sqrt(2.0))))


_HT = 30


def _conv_body(x_ref, a_ref, b_ref, o_ref, xp_ref):
    C, H, W = x_ref.shape[1], x_ref.shape[2], x_ref.shape[3]
    CH, W2 = C * H, 2 * W
    xa = x_ref[0].reshape(CH, W).astype(jnp.bfloat16)
    xb = x_ref[1].reshape(CH, W).astype(jnp.bfloat16)
    zcol = jnp.zeros((CH, 1), jnp.bfloat16)
    zrow = jnp.zeros((C, 1, W2), jnp.bfloat16)
    x = jnp.concatenate([xa, xb], axis=1)
    x_m = jnp.concatenate(
        [zcol, xa[:, : W - 1], zcol, xb[:, : W - 1]], axis=1
    )
    x_p = jnp.concatenate(
        [xa[:, 1:], zcol, xb[:, 1:], zcol], axis=1
    )
    for kx, xs in enumerate((x_m, x, x_p)):
        xp_ref[kx] = jnp.concatenate(
            [zrow, xs.reshape(C, H, W2), zrow], axis=1
        )

    tiles = list(range(0, H - _HT, _HT)) + [H - _HT]
    for h0 in tiles:
        acc = None
        for kx in range(3):
            slab = xp_ref[kx][:, h0 : h0 + _HT + 2, :].reshape(C * (_HT + 2), W2)
            d = jnp.dot(a_ref[kx], slab, preferred_element_type=jnp.float32)
            acc = d if acc is None else acc + d
        y = _gelu_exact(acc.reshape(C, _HT, W2) + b_ref[...][:, None, :])
        o_ref[0, :, h0 : h0 + _HT, :] = y[:, :, :W].astype(o_ref.dtype)
        o_ref[1, :, h0 : h0 + _HT, :] = y[:, :, W:].astype(o_ref.dtype)


def _build_tap_mats(weight_oihw, ht):
    C = weight_oihw.shape[0]
    w = weight_oihw.astype(jnp.float32)
    mats = []
    for kx in range(3):
        m = jnp.zeros((C * ht, C * (ht + 2)), jnp.float32)
        for ky in range(3):
            eye = jnp.eye(ht, ht + 2, k=ky, dtype=jnp.float32)
            m = m + jnp.einsum("oi,dp->odip", w[:, :, ky, kx], eye).reshape(
                C * ht, C * (ht + 2)
            )
        mats.append(m)
    return jnp.stack(mats).astype(jnp.bfloat16)


def _invariant(block_shape):
    index_map = lambda i: (0,) * len(block_shape)
    if hasattr(pl, "Buffered"):
        try:
            return pl.BlockSpec(block_shape, index_map, pipeline_mode=pl.Buffered(1))
        except TypeError:
            pass
    return pl.BlockSpec(block_shape, index_map)


def kernel(x_nchw, weight_oihw, bias):
    N, C, H, W = x_nchw.shape
    a_mats = _build_tap_mats(weight_oihw, _HT)
    bias_mat = jnp.broadcast_to(bias.astype(jnp.float32)[:, None], (C, 2 * W))

    return pl.pallas_call(
        _conv_body,
        out_shape=jax.ShapeDtypeStruct((N, C, H, W), x_nchw.dtype),
        grid=(N // 2,),
        in_specs=[
            pl.BlockSpec((2, C, H, W), lambda i: (i, 0, 0, 0)),
            _invariant((3, C * _HT, C * (_HT + 2))),
            _invariant((C, 2 * W)),
        ],
        out_specs=pl.BlockSpec((2, C, H, W), lambda i: (i, 0, 0, 0)),
        scratch_shapes=[pltpu.VMEM((3, C, H + 2, 2 * W), jnp.bfloat16)],
        compiler_params=pltpu.CompilerParams(
            dimension_semantics=("parallel",),
            vmem_limit_bytes=48 * 1024 * 1024,
        ),
    )(x_nchw, a_mats, bias_mat)


# speedup vs baseline: 10.0591x; 1.2729x over previous
import math

import jax
import jax.numpy as jnp
from jax import lax
from jax.experimental import pallas as pl
from jax.experimental.pallas import tpu as pltpu


def _gelu_exact(x):
    return 0.5 * x * (1.0 + lax.erf(x * (1.0 / math.sqrt(2.0))))


_HT = 24
_HS = 32


def _conv_body(x_ref, a_ref, b_ref, o_ref, xp_ref):
    C, H, W = x_ref.shape[1], x_ref.shape[2], x_ref.shape[3]
    CH, W2 = C * H, 2 * W
    HP = xp_ref.shape[2]
    xa = x_ref[0].reshape(CH, W).astype(jnp.bfloat16)
    xb = x_ref[1].reshape(CH, W).astype(jnp.bfloat16)
    zcol = jnp.zeros((CH, 1), jnp.bfloat16)
    ztop = jnp.zeros((C, 1, W2), jnp.bfloat16)
    zbot = jnp.zeros((C, HP - H - 1, W2), jnp.bfloat16)
    x = jnp.concatenate([xa, xb], axis=1)
    x_m = jnp.concatenate(
        [zcol, xa[:, : W - 1], zcol, xb[:, : W - 1]], axis=1
    )
    x_p = jnp.concatenate(
        [xa[:, 1:], zcol, xb[:, 1:], zcol], axis=1
    )
    for kx, xs in enumerate((x_m, x, x_p)):
        xp_ref[kx] = jnp.concatenate(
            [ztop, xs.reshape(C, H, W2), zbot], axis=1
        )

    for h0 in list(range(0, H - _HT, _HT)) + [H - _HT]:
        slab = xp_ref[:, :, h0 : h0 + _HS, :].reshape(3 * C * _HS, W2)
        acc = jnp.dot(a_ref[...], slab, preferred_element_type=jnp.float32)
        y = _gelu_exact(acc.reshape(C, _HT, W2) + b_ref[...][:, None, :])
        o_ref[0, :, h0 : h0 + _HT, :] = y[:, :, :W].astype(o_ref.dtype)
        o_ref[1, :, h0 : h0 + _HT, :] = y[:, :, W:].astype(o_ref.dtype)


def _build_tap_mats(weight_oihw, ht, hs):
    C = weight_oihw.shape[0]
    w = weight_oihw.astype(jnp.float32)
    mats = []
    for kx in range(3):
        m = jnp.zeros((C * ht, C * hs), jnp.float32)
        for ky in range(3):
            eye = jnp.eye(ht, hs, k=ky, dtype=jnp.float32)
            m = m + jnp.einsum("oi,dp->odip", w[:, :, ky, kx], eye).reshape(
                C * ht, C * hs
            )
        mats.append(m)
    return jnp.concatenate(mats, axis=1).astype(jnp.bfloat16)


def _invariant(block_shape):
    index_map = lambda i: (0,) * len(block_shape)
    if hasattr(pl, "Buffered"):
        try:
            return pl.BlockSpec(block_shape, index_map, pipeline_mode=pl.Buffered(1))
        except TypeError:
            pass
    return pl.BlockSpec(block_shape, index_map)


def kernel(x_nchw, weight_oihw, bias):
    N, C, H, W = x_nchw.shape
    a_mats = _build_tap_mats(weight_oihw, _HT, _HS)
    bias_mat = jnp.broadcast_to(bias.astype(jnp.float32)[:, None], (C, 2 * W))
    hp = H - _HT + _HS + 7
    hp -= hp % 8

    return pl.pallas_call(
        _conv_body,
        out_shape=jax.ShapeDtypeStruct((N, C, H, W), x_nchw.dtype),
        grid=(N // 2,),
        in_specs=[
            pl.BlockSpec((2, C, H, W), lambda i: (i, 0, 0, 0)),
            _invariant((C * _HT, 3 * C * _HS)),
            _invariant((C, 2 * W)),
        ],
        out_specs=pl.BlockSpec((2, C, H, W), lambda i: (i, 0, 0, 0)),
        scratch_shapes=[pltpu.VMEM((3, C, hp, 2 * W), jnp.bfloat16)],
        compiler_params=pltpu.CompilerParams(
            dimension_semantics=("parallel",),
            vmem_limit_bytes=48 * 1024 * 1024,
        ),
    )(x_nchw, a_mats, bias_mat)


# speedup vs baseline: 12.4326x; 1.2360x over previous
import math

import jax
import jax.numpy as jnp
from jax import lax
from jax.experimental import pallas as pl
from jax.experimental.pallas import tpu as pltpu


def _gelu_exact(x):
    return 0.5 * x * (1.0 + lax.erf(x * (1.0 / math.sqrt(2.0))))


_P = 4
_HT = 24
_HS = 32


def _conv_body(x_ref, a_ref, b_ref, o_ref, xp_ref):
    C, H, W = x_ref.shape[1], x_ref.shape[2], x_ref.shape[3]
    CH, WP = C * H, _P * W
    HP = xp_ref.shape[2]
    xs = [x_ref[p].reshape(CH, W).astype(jnp.bfloat16) for p in range(_P)]
    zcol = jnp.zeros((CH, 1), jnp.bfloat16)
    ztop = jnp.zeros((C, 1, WP), jnp.bfloat16)
    zbot = jnp.zeros((C, HP - H - 1, WP), jnp.bfloat16)
    x = jnp.concatenate(xs, axis=1)
    x_m = jnp.concatenate(
        [t for xp in xs for t in (zcol, xp[:, : W - 1])], axis=1
    )
    x_p = jnp.concatenate(
        [t for xp in xs for t in (xp[:, 1:], zcol)], axis=1
    )
    for kx, xk in enumerate((x_m, x, x_p)):
        xp_ref[kx] = jnp.concatenate(
            [ztop, xk.reshape(C, H, WP), zbot], axis=1
        )

    for h0 in list(range(0, H - _HT, _HT)) + [H - _HT]:
        slab = xp_ref[:, :, h0 : h0 + _HS, :].reshape(3 * C * _HS, WP)
        acc = jnp.dot(a_ref[...], slab, preferred_element_type=jnp.float32)
        y = _gelu_exact(acc.reshape(C, _HT, WP) + b_ref[...][:, None, :])
        y = y.astype(o_ref.dtype)
        for p in range(_P):
            o_ref[p, :, h0 : h0 + _HT, :] = y[:, :, p * W : (p + 1) * W]


def _build_tap_mats(weight_oihw, ht, hs):
    C = weight_oihw.shape[0]
    w = weight_oihw.astype(jnp.float32)
    mats = []
    for kx in range(3):
        m = jnp.zeros((C * ht, C * hs), jnp.float32)
        for ky in range(3):
            eye = jnp.eye(ht, hs, k=ky, dtype=jnp.float32)
            m = m + jnp.einsum("oi,dp->odip", w[:, :, ky, kx], eye).reshape(
                C * ht, C * hs
            )
        mats.append(m)
    return jnp.concatenate(mats, axis=1).astype(jnp.bfloat16)


def _invariant(block_shape):
    index_map = lambda i: (0,) * len(block_shape)
    if hasattr(pl, "Buffered"):
        try:
            return pl.BlockSpec(block_shape, index_map, pipeline_mode=pl.Buffered(1))
        except TypeError:
            pass
    return pl.BlockSpec(block_shape, index_map)


def kernel(x_nchw, weight_oihw, bias):
    N, C, H, W = x_nchw.shape
    a_mats = _build_tap_mats(weight_oihw, _HT, _HS)
    bias_mat = jnp.broadcast_to(bias.astype(jnp.float32)[:, None], (C, _P * W))
    hp = H - _HT + _HS + 7
    hp -= hp % 8

    return pl.pallas_call(
        _conv_body,
        out_shape=jax.ShapeDtypeStruct((N, C, H, W), x_nchw.dtype),
        grid=(N // _P,),
        in_specs=[
            pl.BlockSpec((_P, C, H, W), lambda i: (i, 0, 0, 0)),
            _invariant((C * _HT, 3 * C * _HS)),
            _invariant((C, _P * W)),
        ],
        out_specs=pl.BlockSpec((_P, C, H, W), lambda i: (i, 0, 0, 0)),
        scratch_shapes=[pltpu.VMEM((3, C, hp, _P * W), jnp.bfloat16)],
        compiler_params=pltpu.CompilerParams(
            dimension_semantics=("parallel",),
            vmem_limit_bytes=48 * 1024 * 1024,
        ),
    )(x_nchw, a_mats, bias_mat)


# speedup vs baseline: 14.0300x; 1.1285x over previous
import math

import jax
import jax.numpy as jnp
from jax import lax
from jax.experimental import pallas as pl
from jax.experimental.pallas import tpu as pltpu


def _gelu_exact(x):
    return 0.5 * x * (1.0 + lax.erf(x * (1.0 / math.sqrt(2.0))))


_P = 8
_HT = 24
_HS = 32


def _conv_body(x_ref, a_ref, b_ref, o_ref, xp_ref):
    C, H, W = x_ref.shape[1], x_ref.shape[2], x_ref.shape[3]
    CH, WP = C * H, _P * W
    HP = xp_ref.shape[2]
    xs = [x_ref[p].reshape(CH, W).astype(jnp.bfloat16) for p in range(_P)]
    zcol = jnp.zeros((CH, 1), jnp.bfloat16)
    ztop = jnp.zeros((C, 1, WP), jnp.bfloat16)
    zbot = jnp.zeros((C, HP - H - 1, WP), jnp.bfloat16)
    x = jnp.concatenate(xs, axis=1)
    x_m = jnp.concatenate(
        [t for xp in xs for t in (zcol, xp[:, : W - 1])], axis=1
    )
    x_p = jnp.concatenate(
        [t for xp in xs for t in (xp[:, 1:], zcol)], axis=1
    )
    for kx, xk in enumerate((x_m, x, x_p)):
        xp_ref[kx] = jnp.concatenate(
            [ztop, xk.reshape(C, H, WP), zbot], axis=1
        )

    for h0 in list(range(0, H - _HT, _HT)) + [H - _HT]:
        slab = xp_ref[:, :, h0 : h0 + _HS, :].reshape(3 * C * _HS, WP)
        acc = jnp.dot(a_ref[...], slab, preferred_element_type=jnp.float32)
        y = _gelu_exact(acc.reshape(C, _HT, WP) + b_ref[...][:, None, :])
        y = y.astype(o_ref.dtype)
        for p in range(_P):
            o_ref[p, :, h0 : h0 + _HT, :] = y[:, :, p * W : (p + 1) * W]


def _build_tap_mats(weight_oihw, ht, hs):
    C = weight_oihw.shape[0]
    w = weight_oihw.astype(jnp.float32)
    mats = []
    for kx in range(3):
        m = jnp.zeros((C * ht, C * hs), jnp.float32)
        for ky in range(3):
            eye = jnp.eye(ht, hs, k=ky, dtype=jnp.float32)
            m = m + jnp.einsum("oi,dp->odip", w[:, :, ky, kx], eye).reshape(
                C * ht, C * hs
            )
        mats.append(m)
    return jnp.concatenate(mats, axis=1).astype(jnp.bfloat16)


def _invariant(block_shape):
    index_map = lambda i: (0,) * len(block_shape)
    if hasattr(pl, "Buffered"):
        try:
            return pl.BlockSpec(block_shape, index_map, pipeline_mode=pl.Buffered(1))
        except TypeError:
            pass
    return pl.BlockSpec(block_shape, index_map)


def kernel(x_nchw, weight_oihw, bias):
    N, C, H, W = x_nchw.shape
    a_mats = _build_tap_mats(weight_oihw, _HT, _HS)
    bias_mat = jnp.broadcast_to(bias.astype(jnp.float32)[:, None], (C, _P * W))
    hp = H - _HT + _HS + 7
    hp -= hp % 8

    return pl.pallas_call(
        _conv_body,
        out_shape=jax.ShapeDtypeStruct((N, C, H, W), x_nchw.dtype),
        grid=(N // _P,),
        in_specs=[
            pl.BlockSpec((_P, C, H, W), lambda i: (i, 0, 0, 0)),
            _invariant((C * _HT, 3 * C * _HS)),
            _invariant((C, _P * W)),
        ],
        out_specs=pl.BlockSpec((_P, C, H, W), lambda i: (i, 0, 0, 0)),
        scratch_shapes=[pltpu.VMEM((3, C, hp, _P * W), jnp.bfloat16)],
        compiler_params=pltpu.CompilerParams(
            dimension_semantics=("parallel",),
            vmem_limit_bytes=48 * 1024 * 1024,
        ),
    )(x_nchw, a_mats, bias_mat)


# speedup vs baseline: 14.0518x; 1.0016x over previous
import math

import jax
import jax.numpy as jnp
from jax import lax
from jax.experimental import pallas as pl
from jax.experimental.pallas import tpu as pltpu


def _gelu_exact(x):
    return 0.5 * x * (1.0 + lax.erf(x * (1.0 / math.sqrt(2.0))))


_P = 16
_HT = 24
_HS = 32


def _conv_body(x_ref, a_ref, b_ref, o_ref, xp_ref):
    C, H, W = x_ref.shape[1], x_ref.shape[2], x_ref.shape[3]
    CH, WP = C * H, _P * W
    HP = xp_ref.shape[2]
    xs = [x_ref[p].reshape(CH, W).astype(jnp.bfloat16) for p in range(_P)]
    zcol = jnp.zeros((CH, 1), jnp.bfloat16)
    ztop = jnp.zeros((C, 1, WP), jnp.bfloat16)
    zbot = jnp.zeros((C, HP - H - 1, WP), jnp.bfloat16)
    x = jnp.concatenate(xs, axis=1)
    x_m = jnp.concatenate(
        [t for xp in xs for t in (zcol, xp[:, : W - 1])], axis=1
    )
    x_p = jnp.concatenate(
        [t for xp in xs for t in (xp[:, 1:], zcol)], axis=1
    )
    for kx, xk in enumerate((x_m, x, x_p)):
        xp_ref[kx] = jnp.concatenate(
            [ztop, xk.reshape(C, H, WP), zbot], axis=1
        )

    for h0 in list(range(0, H - _HT, _HT)) + [H - _HT]:
        slab = xp_ref[:, :, h0 : h0 + _HS, :].reshape(3 * C * _HS, WP)
        acc = jnp.dot(a_ref[...], slab, preferred_element_type=jnp.float32)
        y = _gelu_exact(acc.reshape(C, _HT, WP) + b_ref[...][:, None, :])
        y = y.astype(o_ref.dtype)
        for p in range(_P):
            o_ref[p, :, h0 : h0 + _HT, :] = y[:, :, p * W : (p + 1) * W]


def _build_tap_mats(weight_oihw, ht, hs):
    C = weight_oihw.shape[0]
    w = weight_oihw.astype(jnp.float32)
    mats = []
    for kx in range(3):
        m = jnp.zeros((C * ht, C * hs), jnp.float32)
        for ky in range(3):
            eye = jnp.eye(ht, hs, k=ky, dtype=jnp.float32)
            m = m + jnp.einsum("oi,dp->odip", w[:, :, ky, kx], eye).reshape(
                C * ht, C * hs
            )
        mats.append(m)
    return jnp.concatenate(mats, axis=1).astype(jnp.bfloat16)


def _invariant(block_shape):
    index_map = lambda i: (0,) * len(block_shape)
    if hasattr(pl, "Buffered"):
        try:
            return pl.BlockSpec(block_shape, index_map, pipeline_mode=pl.Buffered(1))
        except TypeError:
            pass
    return pl.BlockSpec(block_shape, index_map)


def kernel(x_nchw, weight_oihw, bias):
    N, C, H, W = x_nchw.shape
    a_mats = _build_tap_mats(weight_oihw, _HT, _HS)
    bias_mat = jnp.broadcast_to(bias.astype(jnp.float32)[:, None], (C, _P * W))
    hp = H - _HT + _HS + 7
    hp -= hp % 8

    return pl.pallas_call(
        _conv_body,
        out_shape=jax.ShapeDtypeStruct((N, C, H, W), x_nchw.dtype),
        grid=(N // _P,),
        in_specs=[
            pl.BlockSpec((_P, C, H, W), lambda i: (i, 0, 0, 0)),
            _invariant((C * _HT, 3 * C * _HS)),
            _invariant((C, _P * W)),
        ],
        out_specs=pl.BlockSpec((_P, C, H, W), lambda i: (i, 0, 0, 0)),
        scratch_shapes=[pltpu.VMEM((3, C, hp, _P * W), jnp.bfloat16)],
        compiler_params=pltpu.CompilerParams(
            dimension_semantics=("parallel",),
            vmem_limit_bytes=60 * 1024 * 1024,
        ),
    )(x_nchw, a_mats, bias_mat)
